# baseline (speedup 1.0000x reference)
import jax
import jax.numpy as jnp
from jax import lax
from jax.experimental import pallas as pl
from jax.experimental.pallas import tpu as pltpu
from jax.experimental.pallas import tpu_sc as plsc

B = 100000
D = 128
NV = 101
NC = 2
NS = 16
NW = NC * NS
CHUNK = 256
NCHUNK = (B + CHUNK - 1) // CHUNK
LAST_BASE = B - CHUNK
MAX_LOC = (NCHUNK + NW - 1) // NW
NGRP = CHUNK // 16


def _body(z_hbm, tabf_hbm, outf_hbm, ix0, ix1, tab_v, s0, s1,
          p0, p1, w0, w1):
    wid = lax.axis_index("s") * NC + lax.axis_index("c")
    nloc = (NCHUNK - wid + NW - 1) // NW
    idxs = (ix0, ix1)
    isems = (p0, p1)
    stags = (s0, s1)
    wsems = (w0, w1)

    lane = lax.iota(jnp.int32, 16)
    coffs = [lane + c * 16 for c in range(D // 16)]

    def base_of(i):
        cid = wid + i * NW
        return pl.multiple_of(lax.min(cid * CHUNK, LAST_BASE), 8)

    def fire_idx(i, b):
        pltpu.async_copy(
            z_hbm.at[pl.ds(base_of(i), CHUNK)], idxs[b], isems[b]
        )

    def wait_idx(i, b):
        pltpu.make_async_copy(
            z_hbm.at[pl.ds(base_of(i), CHUNK)], idxs[b], isems[b]
        ).wait()

    def compute(i, b):
        @pl.when(i + 1 < nloc)
        def _():
            fire_idx(i + 1, 1 - b)

        wait_idx(i, b)
        idx_v = idxs[b]
        stag = stags[b]

        @plsc.parallel_loop(0, NGRP)
        def _(g):
            zb = idx_v[pl.ds(g * 16, 16)] * D
            gb = g * (16 * D)
            for r in range(16):
                zr = zb.at[jnp.full((16,), r, jnp.int32)].get(
                    mode="promise_in_bounds"
                )
                for c in range(D // 16):
                    vals = plsc.load_gather(tab_v, [zr + coffs[c]])
                    stag[pl.ds(gb + r * D + c * 16, 16)] = vals

    def fire_write(i, b):
        pltpu.async_copy(
            stags[b],
            outf_hbm.at[pl.ds(base_of(i) * D, CHUNK * D)],
            wsems[b],
        )

    def wait_write(i, b):
        pltpu.make_async_copy(
            stags[b],
            outf_hbm.at[pl.ds(base_of(i) * D, CHUNK * D)],
            wsems[b],
        ).wait()

    fire_idx(0, 0)
    pltpu.sync_copy(tabf_hbm, tab_v)

    def step(i, b):
        @pl.when(i < nloc)
        def _():
            @pl.when(i >= 2)
            def _():
                wait_write(i - 2, b)

            compute(i, b)
            fire_write(i, b)

    def loop_body(k, carry):
        step(2 * k, 0)
        step(2 * k + 1, 1)
        return carry

    lax.fori_loop(0, (MAX_LOC + 1) // 2, loop_body, 0)

    even = (nloc % 2) == 0

    @pl.when(even)
    def _():
        wait_write(nloc - 2, 0)
        wait_write(nloc - 1, 1)

    @pl.when(jnp.logical_not(even))
    def _():
        wait_write(nloc - 2, 1)
        wait_write(nloc - 1, 0)


@jax.jit
def kernel(z, table):
    z = z.astype(jnp.int32)
    mesh = plsc.VectorSubcoreMesh(core_axis_name="c", subcore_axis_name="s")
    f = pl.kernel(
        _body,
        out_type=jax.ShapeDtypeStruct((B * D,), jnp.float32),
        mesh=mesh,
        compiler_params=pltpu.CompilerParams(needs_layout_passes=False),
        scratch_types=[
            pltpu.VMEM((CHUNK,), jnp.int32),
            pltpu.VMEM((CHUNK,), jnp.int32),
            pltpu.VMEM((NV * D,), jnp.float32),
            pltpu.VMEM((CHUNK * D,), jnp.float32),
            pltpu.VMEM((CHUNK * D,), jnp.float32),
            pltpu.SemaphoreType.DMA,
            pltpu.SemaphoreType.DMA,
            pltpu.SemaphoreType.DMA,
            pltpu.SemaphoreType.DMA,
        ],
    )
    return f(z, table.reshape(-1)).reshape(B, D)


# speedup vs baseline: 1.1728x; 1.1728x over previous
import jax
import jax.numpy as jnp
from jax import lax
from jax.experimental import pallas as pl
from jax.experimental.pallas import tpu as pltpu
from jax.experimental.pallas import tpu_sc as plsc

B = 100000
D = 128
NV = 101
NC = 2
NS = 16
NW = NC * NS
CHUNK = 416
NCHUNK = (B + CHUNK - 1) // CHUNK
LAST_BASE = B - CHUNK
MAX_LOC = (NCHUNK + NW - 1) // NW
NGRP = CHUNK // 16


def _body(z_hbm, tabf_hbm, outf_hbm, ix0, ix1, tab_v, s0, s1,
          p0, p1, w0, w1):
    wid = lax.axis_index("s") * NC + lax.axis_index("c")
    nloc = (NCHUNK - wid + NW - 1) // NW
    idxs = (ix0, ix1)
    isems = (p0, p1)
    stags = (s0, s1)
    wsems = (w0, w1)

    lane = lax.iota(jnp.int32, 16)
    coffs = [lane + c * 16 for c in range(D // 16)]

    def base_of(i):
        cid = wid + i * NW
        return pl.multiple_of(lax.min(cid * CHUNK, LAST_BASE), 8)

    def fire_idx(i, b):
        pltpu.async_copy(
            z_hbm.at[pl.ds(base_of(i), CHUNK)], idxs[b], isems[b]
        )

    def wait_idx(i, b):
        pltpu.make_async_copy(
            z_hbm.at[pl.ds(base_of(i), CHUNK)], idxs[b], isems[b]
        ).wait()

    def compute(i, b):
        @pl.when(i + 1 < nloc)
        def _():
            fire_idx(i + 1, 1 - b)

        wait_idx(i, b)
        idx_v = idxs[b]
        stag = stags[b]

        @plsc.parallel_loop(0, NGRP)
        def _(g):
            zb = idx_v[pl.ds(g * 16, 16)] * D
            gb = g * (16 * D)
            for r in range(16):
                zr = zb.at[jnp.full((16,), r, jnp.int32)].get(
                    mode="promise_in_bounds"
                )
                for c in range(D // 16):
                    vals = plsc.load_gather(tab_v, [zr + coffs[c]])
                    stag[pl.ds(gb + r * D + c * 16, 16)] = vals

    def fire_write(i, b):
        pltpu.async_copy(
            stags[b],
            outf_hbm.at[pl.ds(base_of(i) * D, CHUNK * D)],
            wsems[b],
        )

    def wait_write(i, b):
        pltpu.make_async_copy(
            stags[b],
            outf_hbm.at[pl.ds(base_of(i) * D, CHUNK * D)],
            wsems[b],
        ).wait()

    fire_idx(0, 0)
    pltpu.sync_copy(tabf_hbm, tab_v)

    def step(i, b):
        @pl.when(i < nloc)
        def _():
            @pl.when(i >= 2)
            def _():
                wait_write(i - 2, b)

            compute(i, b)
            fire_write(i, b)

    def loop_body(k, carry):
        step(2 * k, 0)
        step(2 * k + 1, 1)
        return carry

    lax.fori_loop(0, (MAX_LOC + 1) // 2, loop_body, 0)

    even = (nloc % 2) == 0

    @pl.when(even)
    def _():
        wait_write(nloc - 2, 0)
        wait_write(nloc - 1, 1)

    @pl.when(jnp.logical_not(even))
    def _():
        wait_write(nloc - 2, 1)
        wait_write(nloc - 1, 0)


@jax.jit
def kernel(z, table):
    z = z.astype(jnp.int32)
    mesh = plsc.VectorSubcoreMesh(core_axis_name="c", subcore_axis_name="s")
    f = pl.kernel(
        _body,
        out_type=jax.ShapeDtypeStruct((B * D,), jnp.float32),
        mesh=mesh,
        compiler_params=pltpu.CompilerParams(needs_layout_passes=False),
        scratch_types=[
            pltpu.VMEM((CHUNK,), jnp.int32),
            pltpu.VMEM((CHUNK,), jnp.int32),
            pltpu.VMEM((NV * D,), jnp.float32),
            pltpu.VMEM((CHUNK * D,), jnp.float32),
            pltpu.VMEM((CHUNK * D,), jnp.float32),
            pltpu.SemaphoreType.DMA,
            pltpu.SemaphoreType.DMA,
            pltpu.SemaphoreType.DMA,
            pltpu.SemaphoreType.DMA,
        ],
    )
    return f(z, table.reshape(-1)).reshape(B, D)
